# baseline (speedup 1.0000x reference)
import functools

import jax
import jax.numpy as jnp
from jax import lax
from jax.experimental import pallas as pl
from jax.experimental.pallas import tpu as pltpu
from jax.experimental.pallas import tpu_sc as plsc

_LANES = 16


def _build_kernel(batch, emb_rows, emb_dim):
    info = plsc.get_sparse_core_info()
    num_workers = info.num_cores * info.num_subcores
    assert batch % num_workers == 0
    per_w = batch // num_workers
    chunk = 128 if per_w % 128 == 0 else per_w
    n_chunks = per_w // chunk
    groups = per_w // _LANES

    mesh = plsc.VectorSubcoreMesh(core_axis_name="c", subcore_axis_name="s")

    @functools.partial(
        pl.kernel,
        mesh=mesh,
        out_type=jax.ShapeDtypeStruct((batch, emb_dim), jnp.float32),
        scratch_types=[
            pltpu.VMEM((per_w,), jnp.int32),
            pltpu.VMEM((n_chunks, chunk), jnp.int32),
            pltpu.VMEM((per_w, emb_dim), jnp.float32),
            pltpu.SemaphoreType.DMA,
        ],
        compiler_params=pltpu.CompilerParams(use_tc_tiling_on_sc=False),
    )
    def k(act_hbm, emb_hbm, out_hbm, act_v, idx_v, rows_v, sem):
        wid = lax.axis_index("s") * info.num_cores + lax.axis_index("c")
        base = wid * per_w
        pltpu.sync_copy(act_hbm.at[pl.ds(base, per_w)], act_v)
        for g in range(groups):
            x = act_v[pl.ds(g * _LANES, _LANES)]
            x = jnp.where(x == -1, 0, x)
            x = jnp.where(x == -100, 0, x)
            x = x + 1
            row = (g * _LANES) // chunk
            col = (g * _LANES) % chunk
            idx_v[row, pl.ds(col, _LANES)] = x
        copies = [
            pltpu.async_copy(
                emb_hbm.at[idx_v.at[r]],
                rows_v.at[pl.ds(r * chunk, chunk)],
                sem,
            )
            for r in range(n_chunks)
        ]
        for c in copies:
            c.wait()
        pltpu.sync_copy(rows_v, out_hbm.at[pl.ds(base, per_w)])

    return k


def kernel(action_tuple, action_emb):
    if action_tuple.ndim == 1:
        idx_col = action_tuple
    else:
        idx_col = action_tuple[:, 0]
    batch = idx_col.shape[0]
    emb_rows, emb_dim = action_emb.shape
    k = _build_kernel(batch, emb_rows, emb_dim)
    return k(idx_col.astype(jnp.int32), action_emb)


# speedup vs baseline: 3.7970x; 3.7970x over previous
import functools

import jax
import jax.numpy as jnp
from jax import lax
from jax.experimental import pallas as pl
from jax.experimental.pallas import tpu as pltpu
from jax.experimental.pallas import tpu_sc as plsc

_LANES = 16
_NB = 8
_BANKS = 3


def _build_kernel(batch, emb_dim, emb_rows):
    info = plsc.get_sparse_core_info()
    num_workers = info.num_cores * info.num_subcores
    assert batch % num_workers == 0
    per_w = batch // num_workers
    rounds = per_w // _NB
    span = 2 * _BANKS
    assert rounds % span == 4

    mesh = plsc.VectorSubcoreMesh(core_axis_name="c", subcore_axis_name="s")

    @functools.partial(
        pl.kernel,
        mesh=mesh,
        out_type=jax.ShapeDtypeStruct((emb_dim, batch), jnp.float32),
        scratch_types=[
            pltpu.VMEM((per_w + _NB,), jnp.int32),
            pltpu.VMEM((per_w,), jnp.int32),
            pltpu.VMEM((_BANKS, _NB, emb_dim, 128), jnp.float32),
            pltpu.VMEM((emb_dim, per_w), jnp.float32),
            pltpu.SemaphoreType.DMA,
            pltpu.SemaphoreType.DMA,
            pltpu.SemaphoreType.DMA,
        ],
        compiler_params=pltpu.CompilerParams(needs_layout_passes=False),
    )
    def k(act_hbm, embt_hbm, out_hbm, idx_v, col_v, blk_v, cols_v, s0, s1, s2):
        sems = (s0, s1, s2)
        wid = lax.axis_index("s") * info.num_cores + lax.axis_index("c")
        base = wid * per_w
        lane = lax.iota(jnp.int32, _LANES)
        pltpu.sync_copy(act_hbm.at[pl.ds(base, per_w)], idx_v.at[pl.ds(0, per_w)])
        for g in range(per_w // _LANES):
            x = idx_v[pl.ds(g * _LANES, _LANES)]
            x = jnp.where(x == -1, 0, x)
            x = jnp.where(x == -100, 0, x)
            x = x + 1
            idx_v[pl.ds(g * _LANES, _LANES)] = x
            col_v[pl.ds(g * _LANES, _LANES)] = jnp.bitwise_and(x, 127)

        def fire(r, bank):
            ivec = idx_v[pl.ds(r * _NB, _LANES)]
            for t in range(_NB):
                c0 = pl.multiple_of((ivec[t] // 128) * 128, 128)
                pltpu.async_copy(
                    embt_hbm.at[:, pl.ds(c0, 128)], blk_v.at[bank, t], sems[bank]
                )

        def drain(bank):
            for t in range(_NB):
                pltpu.make_async_copy(
                    embt_hbm.at[:, pl.ds(0, 128)], blk_v.at[bank, t], sems[bank]
                ).wait()

        def extract_pair(r, bank_lo, bank_hi):
            bank_vec = jnp.where(lane < _NB, bank_lo, bank_hi)
            slot_vec = jnp.bitwise_and(lane, _NB - 1)
            col16 = col_v[pl.ds(r * _NB, _LANES)]
            for d in range(emb_dim):
                v = plsc.load_gather(
                    blk_v,
                    [bank_vec, slot_vec, jnp.full((_LANES,), d, jnp.int32), col16],
                )
                cols_v[d, pl.ds(r * _NB, _LANES)] = v

        fire(0, 0)
        fire(1, 1)

        def body(m, _):
            r0 = m * span
            for h in range(span // 2):
                r = r0 + 2 * h
                b0 = (2 * h) % _BANKS
                b1 = (2 * h + 1) % _BANKS
                drain(b0)
                fire(r + 2, (b0 + 2) % _BANKS)
                drain(b1)
                extract_pair(r, b0, b1)
                fire(r + 3, b0)
            return _

        n_loop = (rounds - 4) // span
        lax.fori_loop(0, n_loop, body, None)
        r0 = n_loop * span
        drain(0)
        fire(r0 + 2, 2)
        drain(1)
        extract_pair(r0, 0, 1)
        fire(r0 + 3, 0)
        drain(2)
        drain(0)
        extract_pair(r0 + 2, 2, 0)
        pltpu.sync_copy(cols_v, out_hbm.at[:, pl.ds(base, per_w)])

    return k


def kernel(action_tuple, action_emb):
    if action_tuple.ndim == 1:
        idx_col = action_tuple
    else:
        idx_col = action_tuple[:, 0]
    batch = idx_col.shape[0]
    emb_rows, emb_dim = action_emb.shape
    k = _build_kernel(batch, emb_dim, emb_rows)
    out_t = k(idx_col.astype(jnp.int32), action_emb.T)
    return out_t.T


# speedup vs baseline: 3.8776x; 1.0212x over previous
import functools

import jax
import jax.numpy as jnp
from jax import lax
from jax.experimental import pallas as pl
from jax.experimental.pallas import tpu as pltpu
from jax.experimental.pallas import tpu_sc as plsc

_LANES = 16


def _build_kernel(batch, emb_dim, emb_rows):
    info = plsc.get_sparse_core_info()
    num_workers = info.num_cores * info.num_subcores
    assert batch % num_workers == 0
    per_w = batch // num_workers
    nb = _LANES
    rounds = per_w // nb

    mesh = plsc.VectorSubcoreMesh(core_axis_name="c", subcore_axis_name="s")

    @functools.partial(
        pl.kernel,
        mesh=mesh,
        out_type=jax.ShapeDtypeStruct((emb_dim, batch), jnp.float32),
        scratch_types=[
            pltpu.VMEM((per_w,), jnp.int32),
            pltpu.VMEM((per_w,), jnp.int32),
            pltpu.VMEM((nb, emb_dim, 128), jnp.float32),
            pltpu.VMEM((emb_dim, per_w), jnp.float32),
            pltpu.SemaphoreType.DMA,
        ],
        compiler_params=pltpu.CompilerParams(needs_layout_passes=False),
    )
    def k(act_hbm, embt_hbm, out_hbm, idx_v, col_v, blk_v, cols_v, sem):
        wid = lax.axis_index("s") * info.num_cores + lax.axis_index("c")
        base = wid * per_w
        lane = lax.iota(jnp.int32, _LANES)
        pltpu.sync_copy(act_hbm.at[pl.ds(base, per_w)], idx_v)
        for g in range(per_w // _LANES):
            x = idx_v[pl.ds(g * _LANES, _LANES)]
            x = jnp.where(x == -1, 0, x)
            x = jnp.where(x == -100, 0, x)
            x = x + 1
            idx_v[pl.ds(g * _LANES, _LANES)] = x
            col_v[pl.ds(g * _LANES, _LANES)] = jnp.bitwise_and(x, 127)

        def round_body(r, _):
            ivec = idx_v[pl.ds(r * nb, _LANES)]
            copies = []
            for t in range(nb):
                c0 = pl.multiple_of((ivec[t] // 128) * 128, 128)
                copies.append(
                    pltpu.async_copy(
                        embt_hbm.at[:, pl.ds(c0, 128)], blk_v.at[t], sem
                    )
                )
            for c in copies:
                c.wait()
            col16 = col_v[pl.ds(r * nb, _LANES)]
            for d in range(emb_dim):
                v = plsc.load_gather(
                    blk_v, [lane, jnp.full((_LANES,), d, jnp.int32), col16]
                )
                cols_v[d, pl.ds(r * nb, _LANES)] = v
            return _

        lax.fori_loop(0, rounds, round_body, None)
        pltpu.sync_copy(cols_v, out_hbm.at[:, pl.ds(base, per_w)])

    return k


def kernel(action_tuple, action_emb):
    if action_tuple.ndim == 1:
        idx_col = action_tuple
    else:
        idx_col = action_tuple[:, 0]
    batch = idx_col.shape[0]
    emb_rows, emb_dim = action_emb.shape
    k = _build_kernel(batch, emb_dim, emb_rows)
    out_t = k(idx_col.astype(jnp.int32), action_emb.T)
    return out_t.T
